# baseline (speedup 1.0000x reference)
import jax
import jax.numpy as jnp
from jax import lax
from jax.experimental import pallas as pl
from jax.experimental.pallas import tpu as pltpu
from jax.experimental.pallas import tpu_sc as plsc

N_NODES = 10000
D = 128
N_EDGES = 320000
HID = 32
L = 16

NC = 2
NS = 16
NW = NC * NS
EDGES_PER_W = N_EDGES // NW
CHUNK = 400
N_CHUNKS = EDGES_PER_W // CHUNK
N_SUB = 4
SUB = CHUNK // N_SUB
N_GROUPS = CHUNK // L
N_PAIRS = (N_CHUNKS - 1) // 2


def _proj_body(u_ref, i_ref, wu_ref, wi_ref, b1_ref, pu_ref, pi_ref):
    pu_ref[...] = lax.dot_general(
        u_ref[...], wu_ref[...], (((1,), (1,)), ((), ())),
        preferred_element_type=jnp.float32) + b1_ref[...]
    pi_ref[...] = lax.dot_general(
        i_ref[...], wi_ref[...], (((1,), (1,)), ((), ())),
        preferred_element_type=jnp.float32)


def _project(u_emb, i_emb, w1u, w1i, b1):
    bm = 2000
    grid = (N_NODES // bm,)
    return pl.pallas_call(
        _proj_body,
        grid=grid,
        in_specs=[
            pl.BlockSpec((bm, D), lambda m: (m, 0)),
            pl.BlockSpec((bm, D), lambda m: (m, 0)),
            pl.BlockSpec((HID, D), lambda m: (0, 0)),
            pl.BlockSpec((HID, D), lambda m: (0, 0)),
            pl.BlockSpec((1, HID), lambda m: (0, 0)),
        ],
        out_specs=[
            pl.BlockSpec((bm, HID), lambda m: (m, 0)),
            pl.BlockSpec((bm, HID), lambda m: (m, 0)),
        ],
        out_shape=[
            jax.ShapeDtypeStruct((N_NODES, HID), jnp.float32),
            jax.ShapeDtypeStruct((N_NODES, HID), jnp.float32),
        ],
    )(u_emb, i_emb, w1u, w1i, b1)


def _sc_body(pu_hbm, pi_hbm, uidx_hbm, iidx_hbm, w2_hbm, b2_hbm, out_hbm,
             uidx_v, iidx_v, bufu0, bufi0, bufu1, bufi1, logit_v,
             w2t_v, b2_v, pu_sh, pi_sh, semA, semB):
    sid = lax.axis_index("s")
    wid = sid * NC + lax.axis_index("c")
    rows = N_NODES // NS
    pltpu.sync_copy(pu_hbm.at[pl.ds(sid * rows, rows)],
                    pu_sh.at[pl.ds(sid * rows, rows)])
    pltpu.sync_copy(pi_hbm.at[pl.ds(sid * rows, rows)],
                    pi_sh.at[pl.ds(sid * rows, rows)])
    pltpu.sync_copy(w2_hbm, w2t_v)
    pltpu.sync_copy(b2_hbm, b2_v)
    pltpu.sync_copy(uidx_hbm.at[wid], uidx_v)
    pltpu.sync_copy(iidx_hbm.at[wid], iidx_v)
    plsc.subcore_barrier()

    b2s = b2_v[...]
    idx_base = lax.iota(jnp.int32, L)

    def fire(c, bufu, bufi, sem):
        for s in range(N_SUB):
            pltpu.async_copy(
                pu_sh.at[uidx_v.at[c, s]], bufu.at[pl.ds(s * SUB, SUB)], sem)
            pltpu.async_copy(
                pi_sh.at[iidx_v.at[c, s]], bufi.at[pl.ds(s * SUB, SUB)], sem)

    def drain(bufu, bufi, sem):
        pltpu.make_async_copy(pu_hbm.at[pl.ds(0, CHUNK)], bufu, sem).wait()
        pltpu.make_async_copy(pi_hbm.at[pl.ds(0, CHUNK)], bufi, sem).wait()

    def compute(c, bufu, bufi):
        def group_body(g, carry):
            base_row = g * L
            rows_g = idx_base + base_row
            acc = b2s
            for k in range(HID):
                ck = jnp.full((L,), k, jnp.int32)
                vu = plsc.load_gather(bufu, [rows_g, ck])
                vi = plsc.load_gather(bufi, [rows_g, ck])
                s = vu + vi
                h = jnp.where(s >= 0, s, 0.2 * s)
                acc = acc + h * w2t_v[k, :]
            sig = 1.0 / (1.0 + jnp.exp(-acc))
            logit_v[pl.ds(c * CHUNK + base_row, L)] = sig
            return carry

        lax.fori_loop(0, N_GROUPS, group_body, 0, unroll=False)

    fire(0, bufu0, bufi0, semA)

    def pair_body(i, carry):
        cA = 2 * i
        fire(cA + 1, bufu1, bufi1, semB)
        drain(bufu0, bufi0, semA)
        compute(cA, bufu0, bufi0)
        fire(cA + 2, bufu0, bufi0, semA)
        drain(bufu1, bufi1, semB)
        compute(cA + 1, bufu1, bufi1)
        return carry

    lax.fori_loop(0, N_PAIRS, pair_body, 0, unroll=False)

    drain(bufu0, bufi0, semA)
    compute(N_CHUNKS - 1, bufu0, bufi0)

    pltpu.sync_copy(logit_v, out_hbm.at[pl.ds(wid * EDGES_PER_W, EDGES_PER_W)])


_SC_CACHE = {}


def _sc_edge_call():
    if "k" not in _SC_CACHE:
        _SC_CACHE["k"] = pl.kernel(
            _sc_body,
            out_type=jax.ShapeDtypeStruct((N_EDGES,), jnp.float32),
            mesh=plsc.VectorSubcoreMesh(
                core_axis_name="c", subcore_axis_name="s",
                num_cores=NC, num_subcores=NS),
            compiler_params=pltpu.CompilerParams(
                use_tc_tiling_on_sc=False, needs_layout_passes=False),
            scratch_types=[
                pltpu.VMEM((N_CHUNKS, N_SUB, SUB), jnp.int32),
                pltpu.VMEM((N_CHUNKS, N_SUB, SUB), jnp.int32),
                pltpu.VMEM((CHUNK, HID), jnp.float32),
                pltpu.VMEM((CHUNK, HID), jnp.float32),
                pltpu.VMEM((CHUNK, HID), jnp.float32),
                pltpu.VMEM((CHUNK, HID), jnp.float32),
                pltpu.VMEM((EDGES_PER_W,), jnp.float32),
                pltpu.VMEM((HID, L), jnp.float32),
                pltpu.VMEM((L,), jnp.float32),
                pltpu.VMEM_SHARED((N_NODES, HID), jnp.float32),
                pltpu.VMEM_SHARED((N_NODES, HID), jnp.float32),
                pltpu.SemaphoreType.DMA,
                pltpu.SemaphoreType.DMA,
            ],
        )
    return _SC_CACHE["k"]


def kernel(user_embeddings, item_embeddings, edge_index, W1, b1, W2, b2):
    ei = edge_index.astype(jnp.int32)
    u_idx = ei[0].reshape(NW, N_CHUNKS, N_SUB, SUB)
    i_idx = ei[1].reshape(NW, N_CHUNKS, N_SUB, SUB)
    w1u = W1[:, :D]
    w1i = W1[:, D:]
    pu, pi = _project(user_embeddings, item_embeddings, w1u, w1i,
                      b1.reshape(1, HID))
    w2t = jnp.broadcast_to(W2.reshape(HID, 1), (HID, L))
    b2s = jnp.broadcast_to(b2.reshape(1), (L,))
    return _sc_edge_call()(pu, pi, u_idx, i_idx, w2t, b2s)


# speedup vs baseline: 3.3885x; 3.3885x over previous
import jax
import jax.numpy as jnp
from jax import lax
from jax.experimental import pallas as pl
from jax.experimental.pallas import tpu as pltpu
from jax.experimental.pallas import tpu_sc as plsc

N_NODES = 10000
D = 128
N_EDGES = 320000
HID = 32
L = 16

NC = 2
NS = 16
NW = NC * NS
EDGES_PER_W = N_EDGES // NW
CHUNK = 400
N_CHUNKS = EDGES_PER_W // CHUNK
N_SUB = 4
SUB = CHUNK // N_SUB
N_GROUPS = CHUNK // L
N_PAIRS = (N_CHUNKS - 1) // 2


def _proj_body(u_ref, i_ref, wu_ref, wi_ref, b1_ref, pu_ref, pi_ref):
    pu_ref[...] = lax.dot_general(
        u_ref[...], wu_ref[...], (((1,), (1,)), ((), ())),
        preferred_element_type=jnp.float32) + b1_ref[...]
    pi_ref[...] = lax.dot_general(
        i_ref[...], wi_ref[...], (((1,), (1,)), ((), ())),
        preferred_element_type=jnp.float32)


def _project(u_emb, i_emb, w1u, w1i, b1):
    bm = 2000
    grid = (N_NODES // bm,)
    return pl.pallas_call(
        _proj_body,
        grid=grid,
        in_specs=[
            pl.BlockSpec((bm, D), lambda m: (m, 0)),
            pl.BlockSpec((bm, D), lambda m: (m, 0)),
            pl.BlockSpec((HID, D), lambda m: (0, 0)),
            pl.BlockSpec((HID, D), lambda m: (0, 0)),
            pl.BlockSpec((1, HID), lambda m: (0, 0)),
        ],
        out_specs=[
            pl.BlockSpec((bm, HID), lambda m: (m, 0)),
            pl.BlockSpec((bm, HID), lambda m: (m, 0)),
        ],
        out_shape=[
            jax.ShapeDtypeStruct((N_NODES, HID), jnp.float32),
            jax.ShapeDtypeStruct((N_NODES, HID), jnp.float32),
        ],
    )(u_emb, i_emb, w1u, w1i, b1)


def _sc_body(pu_hbm, pi_hbm, uidx_hbm, iidx_hbm, w2_hbm, b2_hbm, out_hbm,
             uidx_v, iidx_v, bufu0, bufi0, bufu1, bufi1, logit_v,
             w2_v, b2_v, semA, semB):
    wid = lax.axis_index("s") * NC + lax.axis_index("c")
    pltpu.sync_copy(w2_hbm, w2_v)
    pltpu.sync_copy(b2_hbm, b2_v)
    pltpu.sync_copy(uidx_hbm.at[wid], uidx_v)
    pltpu.sync_copy(iidx_hbm.at[wid], iidx_v)

    w2a = w2_v[pl.ds(0, L)]
    w2b = w2_v[pl.ds(L, L)]
    b2s = b2_v[...]
    lane = lax.iota(jnp.int32, L)

    def fire(c, bufu, bufi, sem):
        for s in range(N_SUB):
            pltpu.async_copy(
                pu_hbm.at[uidx_v.at[c, s]], bufu.at[pl.ds(s * SUB, SUB)], sem)
            pltpu.async_copy(
                pi_hbm.at[iidx_v.at[c, s]], bufi.at[pl.ds(s * SUB, SUB)], sem)

    def drain(bufu, bufi, sem):
        pltpu.make_async_copy(pu_hbm.at[pl.ds(0, CHUNK)], bufu, sem).wait()
        pltpu.make_async_copy(pi_hbm.at[pl.ds(0, CHUNK)], bufi, sem).wait()

    def compute(c, bufu, bufi):
        def group_body(g, carry):
            base_row = g * L
            acc = jnp.zeros((L,), jnp.float32)
            for j in range(L):
                r = base_row + j
                s0 = bufu[r, pl.ds(0, L)] + bufi[r, pl.ds(0, L)]
                s1 = bufu[r, pl.ds(L, L)] + bufi[r, pl.ds(L, L)]
                h0 = jnp.maximum(s0, 0.2 * s0)
                h1 = jnp.maximum(s1, 0.2 * s1)
                m = h0 * w2a + h1 * w2b
                acc = jnp.where(lane == j, jnp.sum(m), acc)
            x = acc + b2s
            sig = 1.0 / (1.0 + jnp.exp(-x))
            logit_v[pl.ds(c * CHUNK + base_row, L)] = sig
            return carry

        lax.fori_loop(0, N_GROUPS, group_body, 0, unroll=2)

    fire(0, bufu0, bufi0, semA)

    def pair_body(i, carry):
        cA = 2 * i
        fire(cA + 1, bufu1, bufi1, semB)
        drain(bufu0, bufi0, semA)
        compute(cA, bufu0, bufi0)
        fire(cA + 2, bufu0, bufi0, semA)
        drain(bufu1, bufi1, semB)
        compute(cA + 1, bufu1, bufi1)
        return carry

    lax.fori_loop(0, N_PAIRS, pair_body, 0, unroll=False)

    drain(bufu0, bufi0, semA)
    compute(N_CHUNKS - 1, bufu0, bufi0)

    pltpu.sync_copy(logit_v, out_hbm.at[pl.ds(wid * EDGES_PER_W, EDGES_PER_W)])


_SC_CACHE = {}


def _sc_edge_call():
    if "k" not in _SC_CACHE:
        _SC_CACHE["k"] = pl.kernel(
            _sc_body,
            out_type=jax.ShapeDtypeStruct((N_EDGES,), jnp.float32),
            mesh=plsc.VectorSubcoreMesh(
                core_axis_name="c", subcore_axis_name="s",
                num_cores=NC, num_subcores=NS),
            compiler_params=pltpu.CompilerParams(
                use_tc_tiling_on_sc=False, needs_layout_passes=False),
            scratch_types=[
                pltpu.VMEM((N_CHUNKS, N_SUB, SUB), jnp.int32),
                pltpu.VMEM((N_CHUNKS, N_SUB, SUB), jnp.int32),
                pltpu.VMEM((CHUNK, HID), jnp.float32),
                pltpu.VMEM((CHUNK, HID), jnp.float32),
                pltpu.VMEM((CHUNK, HID), jnp.float32),
                pltpu.VMEM((CHUNK, HID), jnp.float32),
                pltpu.VMEM((EDGES_PER_W,), jnp.float32),
                pltpu.VMEM((HID,), jnp.float32),
                pltpu.VMEM((L,), jnp.float32),
                pltpu.SemaphoreType.DMA,
                pltpu.SemaphoreType.DMA,
            ],
        )
    return _SC_CACHE["k"]


def kernel(user_embeddings, item_embeddings, edge_index, W1, b1, W2, b2):
    ei = edge_index.astype(jnp.int32)
    u_idx = ei[0].reshape(NW, N_CHUNKS, N_SUB, SUB)
    i_idx = ei[1].reshape(NW, N_CHUNKS, N_SUB, SUB)
    w1u = W1[:, :D]
    w1i = W1[:, D:]
    pu, pi = _project(user_embeddings, item_embeddings, w1u, w1i,
                      b1.reshape(1, HID))
    w2 = W2.reshape(HID)
    b2s = jnp.broadcast_to(b2.reshape(1), (L,))
    return _sc_edge_call()(pu, pi, u_idx, i_idx, w2, b2s)
